# baseline (speedup 1.0000x reference)
import jax
import jax.numpy as jnp
from jax import lax
from jax.experimental import pallas as pl
from jax.experimental.pallas import tpu as pltpu
from jax.experimental.pallas import tpu_sc as plsc

NUM_CLASSES = 26
FEATURE_DIM = 128
BATCH = 16384
NC, NS, L = 2, 16, 16
NW = NC * NS
ROWS_W = BATCH // NW
NSEG = FEATURE_DIM // L
ACC_ROWS = 32
CHUNK = 128
NCH = ROWS_W // CHUNK


def _seg_body(feat_hbm, lab_hbm, out_hbm,
              f0, f1, f2, f3, lab_v, zbuf, acc_sh, s0, s1, s2, s3):
    cid = lax.axis_index("c")
    sid = lax.axis_index("s")
    wid = sid * NC + cid
    rbase = wid * ROWS_W
    bufs = (f0, f1, f2, f3)
    sems = (s0, s1, s2, s3)
    copies = [
        pltpu.async_copy(feat_hbm.at[pl.ds(rbase + c * CHUNK, CHUNK)],
                         bufs[c], sems[c])
        for c in range(NCH)
    ]
    for c in range(NCH):
        pltpu.sync_copy(lab_hbm.at[pl.ds(rbase + c * CHUNK, CHUNK)],
                        lab_v.at[c])

    zeros = jnp.zeros((L,), jnp.float32)

    @pl.when(sid == 0)
    def _():
        @plsc.parallel_loop(0, ACC_ROWS)
        def _zero(r):
            for j in range(NSEG):
                zbuf[r, pl.ds(j * L, L)] = zeros
        pltpu.sync_copy(zbuf, acc_sh)

    plsc.subcore_barrier()

    for c in range(NCH):
        copies[c].wait()
        pltpu.sync_copy(bufs[c], acc_sh.at[lab_v.at[c]], add=True)

    plsc.subcore_barrier()

    @pl.when(sid == 0)
    def _():
        pltpu.sync_copy(acc_sh, out_hbm.at[pl.ds(cid * ACC_ROWS, ACC_ROWS)])


_seg_sum = pl.kernel(
    _seg_body,
    out_type=jax.ShapeDtypeStruct((NC * ACC_ROWS, FEATURE_DIM), jnp.float32),
    mesh=plsc.VectorSubcoreMesh(core_axis_name="c", subcore_axis_name="s",
                                num_cores=NC, num_subcores=NS),
    compiler_params=pltpu.CompilerParams(needs_layout_passes=False),
    scratch_types=(
        [pltpu.VMEM((CHUNK, FEATURE_DIM), jnp.float32) for _ in range(NCH)]
        + [pltpu.VMEM((NCH, CHUNK), jnp.int32),
           pltpu.VMEM((ACC_ROWS, FEATURE_DIM), jnp.float32),
           pltpu.VMEM_SHARED((ACC_ROWS, FEATURE_DIM), jnp.float32)]
        + [pltpu.SemaphoreType.DMA for _ in range(NCH)]
    ),
)


def _fin_body(part_ref, cen_ref, rule_ref, out_ref):
    s_full = part_ref[0:ACC_ROWS, :]
    for w in range(1, NC):
        s_full = s_full + part_ref[w * ACC_ROWS:(w + 1) * ACC_ROWS, :]
    s = s_full[:NUM_CLASSES, :]

    cen = cen_ref[...]
    norms = jnp.sqrt(jnp.sum(cen * cen, axis=1, keepdims=True))
    nc = cen / jnp.maximum(norms, 1e-12)
    cos_sum = jnp.sum(s * nc)

    n = NUM_CLASSES
    r0 = jax.lax.broadcasted_iota(jnp.int32, (n, n), 0)
    r1 = jax.lax.broadcasted_iota(jnp.int32, (n, n), 1)
    sim_w = jnp.where(r0 == r1, 0.0, rule_ref[...])
    wsum = jnp.sum(sim_w, axis=1, keepdims=True) + 1e-8
    nw = sim_w / wsum
    expected = jax.lax.dot_general(nw, nc, (((1,), (0,)), ((), ())),
                                   preferred_element_type=jnp.float32)
    loss_reg = 1.0 - jnp.sum(nc * expected) / n
    loss_center = 1.0 - cos_sum / BATCH
    out_ref[...] = jnp.reshape(loss_center + 0.5 * loss_reg, (1, 1))


def kernel(features, labels, centers, rule_matrix):
    partials = _seg_sum(features, labels.astype(jnp.int32))
    out = pl.pallas_call(
        _fin_body,
        out_shape=jax.ShapeDtypeStruct((1, 1), jnp.float32),
    )(partials, centers, rule_matrix)
    return out[0, 0]


# speedup vs baseline: 1.0011x; 1.0011x over previous
import jax
import jax.numpy as jnp
from jax import lax
from jax.experimental import pallas as pl
from jax.experimental.pallas import tpu as pltpu
from jax.experimental.pallas import tpu_sc as plsc

NUM_CLASSES = 26
FEATURE_DIM = 128
BATCH = 16384
NC, NS, L = 2, 16, 16
NW = NC * NS
ROWS_W = BATCH // NW
NSEG = FEATURE_DIM // L
ACC_ROWS = 32
CHUNK = 128
NCH = ROWS_W // CHUNK


def _seg_body(feat_hbm, lab_hbm, out_hbm,
              f0, f1, f2, f3, lab_v, zbuf, acc_sh,
              s0, s1, s2, s3, a0, a1, a2, a3):
    cid = lax.axis_index("c")
    sid = lax.axis_index("s")
    wid = sid * NC + cid
    rbase = wid * ROWS_W
    bufs = (f0, f1, f2, f3)
    sems = (s0, s1, s2, s3)
    a_sems = (a0, a1, a2, a3)
    copies = [
        pltpu.async_copy(feat_hbm.at[pl.ds(rbase + c * CHUNK, CHUNK)],
                         bufs[c], sems[c])
        for c in range(NCH)
    ]
    for c in range(NCH):
        pltpu.sync_copy(lab_hbm.at[pl.ds(rbase + c * CHUNK, CHUNK)],
                        lab_v.at[c])

    zeros = jnp.zeros((L,), jnp.float32)

    @pl.when(sid == 0)
    def _():
        @plsc.parallel_loop(0, ACC_ROWS)
        def _zero(r):
            for j in range(NSEG):
                zbuf[r, pl.ds(j * L, L)] = zeros
        pltpu.sync_copy(zbuf, acc_sh)

    plsc.subcore_barrier()

    adds = []
    for c in range(NCH):
        copies[c].wait()
        adds.append(pltpu.async_copy(bufs[c], acc_sh.at[lab_v.at[c]],
                                     a_sems[c], add=True))
    for a in adds:
        a.wait()

    plsc.subcore_barrier()

    @pl.when(sid == 0)
    def _():
        pltpu.sync_copy(acc_sh, out_hbm.at[pl.ds(cid * ACC_ROWS, ACC_ROWS)])


_seg_sum = pl.kernel(
    _seg_body,
    out_type=jax.ShapeDtypeStruct((NC * ACC_ROWS, FEATURE_DIM), jnp.float32),
    mesh=plsc.VectorSubcoreMesh(core_axis_name="c", subcore_axis_name="s",
                                num_cores=NC, num_subcores=NS),
    compiler_params=pltpu.CompilerParams(needs_layout_passes=False),
    scratch_types=(
        [pltpu.VMEM((CHUNK, FEATURE_DIM), jnp.float32) for _ in range(NCH)]
        + [pltpu.VMEM((NCH, CHUNK), jnp.int32),
           pltpu.VMEM((ACC_ROWS, FEATURE_DIM), jnp.float32),
           pltpu.VMEM_SHARED((ACC_ROWS, FEATURE_DIM), jnp.float32)]
        + [pltpu.SemaphoreType.DMA for _ in range(2 * NCH)]
    ),
)


def _fin_body(part_ref, cen_ref, rule_ref, out_ref):
    s_full = part_ref[0:ACC_ROWS, :]
    for w in range(1, NC):
        s_full = s_full + part_ref[w * ACC_ROWS:(w + 1) * ACC_ROWS, :]
    s = s_full[:NUM_CLASSES, :]

    cen = cen_ref[...]
    norms = jnp.sqrt(jnp.sum(cen * cen, axis=1, keepdims=True))
    nc = cen / jnp.maximum(norms, 1e-12)
    cos_sum = jnp.sum(s * nc)

    n = NUM_CLASSES
    r0 = jax.lax.broadcasted_iota(jnp.int32, (n, n), 0)
    r1 = jax.lax.broadcasted_iota(jnp.int32, (n, n), 1)
    sim_w = jnp.where(r0 == r1, 0.0, rule_ref[...])
    wsum = jnp.sum(sim_w, axis=1, keepdims=True) + 1e-8
    nw = sim_w / wsum
    expected = jax.lax.dot_general(nw, nc, (((1,), (0,)), ((), ())),
                                   preferred_element_type=jnp.float32)
    loss_reg = 1.0 - jnp.sum(nc * expected) / n
    loss_center = 1.0 - cos_sum / BATCH
    out_ref[...] = jnp.reshape(loss_center + 0.5 * loss_reg, (1, 1))


def kernel(features, labels, centers, rule_matrix):
    partials = _seg_sum(features, labels.astype(jnp.int32))
    out = pl.pallas_call(
        _fin_body,
        out_shape=jax.ShapeDtypeStruct((1, 1), jnp.float32),
    )(partials, centers, rule_matrix)
    return out[0, 0]


# speedup vs baseline: 1.0511x; 1.0499x over previous
import jax
import jax.numpy as jnp
from jax import lax
from jax.experimental import pallas as pl
from jax.experimental.pallas import tpu as pltpu
from jax.experimental.pallas import tpu_sc as plsc

NUM_CLASSES = 26
FEATURE_DIM = 128
BATCH = 16384
NC, NS, L = 2, 16, 16
NW = NC * NS
ROWS_W = BATCH // NW
NSEG = FEATURE_DIM // L
ACC_ROWS = 32
CHUNK = 128
NCH = ROWS_W // CHUNK


def _seg_body(feat_hbm, lab_hbm, out_hbm,
              f0, f1, f2, f3, lab_v, zbuf, acc_sh,
              s0, s1, s2, s3, a0, a1, a2, a3):
    cid = lax.axis_index("c")
    sid = lax.axis_index("s")
    wid = sid * NC + cid
    rbase = wid * ROWS_W
    bufs = (f0, f1, f2, f3)
    sems = (s0, s1, s2, s3)
    a_sems = (a0, a1, a2, a3)
    copies = [
        pltpu.async_copy(feat_hbm.at[pl.ds(rbase + c * CHUNK, CHUNK)],
                         bufs[c], sems[c])
        for c in range(NCH)
    ]
    lab_copies = [
        pltpu.async_copy(lab_hbm.at[pl.ds(rbase + c * CHUNK, CHUNK)],
                         lab_v.at[c], a_sems[c])
        for c in range(NCH)
    ]

    zeros = jnp.zeros((L,), jnp.float32)

    @pl.when(sid == 0)
    def _():
        @plsc.parallel_loop(0, ACC_ROWS)
        def _zero(r):
            for j in range(NSEG):
                zbuf[r, pl.ds(j * L, L)] = zeros
        pltpu.sync_copy(zbuf, acc_sh)

    plsc.subcore_barrier()

    for lc in lab_copies:
        lc.wait()

    adds = []
    for c in range(NCH):
        copies[c].wait()
        adds.append(pltpu.async_copy(bufs[c], acc_sh.at[lab_v.at[c]],
                                     a_sems[c], add=True))
    for a in adds:
        a.wait()

    plsc.subcore_barrier()

    @pl.when(sid == 0)
    def _():
        pltpu.sync_copy(acc_sh, out_hbm.at[pl.ds(cid * ACC_ROWS, ACC_ROWS)])


_seg_sum = pl.kernel(
    _seg_body,
    out_type=jax.ShapeDtypeStruct((NC * ACC_ROWS, FEATURE_DIM), jnp.float32),
    mesh=plsc.VectorSubcoreMesh(core_axis_name="c", subcore_axis_name="s",
                                num_cores=NC, num_subcores=NS),
    compiler_params=pltpu.CompilerParams(needs_layout_passes=False),
    scratch_types=(
        [pltpu.VMEM((CHUNK, FEATURE_DIM), jnp.float32) for _ in range(NCH)]
        + [pltpu.VMEM((NCH, CHUNK), jnp.int32),
           pltpu.VMEM((ACC_ROWS, FEATURE_DIM), jnp.float32),
           pltpu.VMEM_SHARED((ACC_ROWS, FEATURE_DIM), jnp.float32)]
        + [pltpu.SemaphoreType.DMA for _ in range(2 * NCH)]
    ),
)


def _fin_body(part_ref, cen_ref, rule_ref, out_ref):
    s_full = part_ref[0:ACC_ROWS, :]
    for w in range(1, NC):
        s_full = s_full + part_ref[w * ACC_ROWS:(w + 1) * ACC_ROWS, :]
    s = s_full[:NUM_CLASSES, :]

    cen = cen_ref[...]
    norms = jnp.sqrt(jnp.sum(cen * cen, axis=1, keepdims=True))
    nc = cen / jnp.maximum(norms, 1e-12)
    cos_sum = jnp.sum(s * nc)

    n = NUM_CLASSES
    r0 = jax.lax.broadcasted_iota(jnp.int32, (n, n), 0)
    r1 = jax.lax.broadcasted_iota(jnp.int32, (n, n), 1)
    sim_w = jnp.where(r0 == r1, 0.0, rule_ref[...])
    wsum = jnp.sum(sim_w, axis=1, keepdims=True) + 1e-8
    nw = sim_w / wsum
    expected = jax.lax.dot_general(nw, nc, (((1,), (0,)), ((), ())),
                                   preferred_element_type=jnp.float32)
    loss_reg = 1.0 - jnp.sum(nc * expected) / n
    loss_center = 1.0 - cos_sum / BATCH
    out_ref[...] = jnp.reshape(loss_center + 0.5 * loss_reg, (1, 1))


def kernel(features, labels, centers, rule_matrix):
    partials = _seg_sum(features, labels.astype(jnp.int32))
    out = pl.pallas_call(
        _fin_body,
        out_shape=jax.ShapeDtypeStruct((1, 1), jnp.float32),
    )(partials, centers, rule_matrix)
    return out[0, 0]
